# Initial kernel scaffold; baseline (speedup 1.0000x reference)
import jax
import jax.numpy as jnp
from jax.experimental import pallas as pl


def kernel(x, edge_index, W, b):
    raise NotImplementedError("write your pallas kernel here")


# speedup vs baseline: 13.3456x; 13.3456x over previous
import functools

import jax
import jax.numpy as jnp
from jax import lax
from jax.experimental import pallas as pl
from jax.experimental.pallas import tpu as pltpu
from jax.experimental.pallas import tpu_sc as plsc

NC = 2
NS = 16
NW = NC * NS
K = 128
BR = 512
DEGW = 128


def _sc_degree(dst_pad, zerosW, onesW, NP, EPW, CH, degw=DEGW):
    mesh = plsc.VectorSubcoreMesh(
        core_axis_name="c", subcore_axis_name="s", num_cores=NC, num_subcores=NS)
    rpt = NP // NS

    @functools.partial(
        pl.kernel,
        out_type=jax.ShapeDtypeStruct((NC, NP, degw), jnp.float32),
        mesh=mesh,
        scratch_types=[
            pltpu.VMEM((K,), jnp.int32),
            pltpu.VMEM((K, degw), jnp.float32),
            pltpu.VMEM_SHARED((NP, degw), jnp.float32),
        ],
    )
    def k(dst_hbm, zeros_hbm, ones_hbm, out_hbm, didx_v, ones_v, acc_sh):
        c = lax.axis_index("c")
        s = lax.axis_index("s")
        wid = s * NC + c
        pltpu.sync_copy(zeros_hbm.at[pl.ds(s * rpt, rpt)],
                        acc_sh.at[pl.ds(s * rpt, rpt)])
        pltpu.sync_copy(ones_hbm, ones_v)
        plsc.subcore_barrier()
        e0 = pl.multiple_of(wid * EPW, 8)

        def step(i, carry):
            off = pl.multiple_of(e0 + i * K, 8)
            pltpu.sync_copy(dst_hbm.at[pl.ds(off, K)], didx_v)
            pltpu.sync_copy(ones_v, acc_sh.at[didx_v], add=True)
            return carry

        lax.fori_loop(0, CH, step, 0)
        plsc.subcore_barrier()
        pltpu.sync_copy(acc_sh.at[pl.ds(s * rpt, rpt)],
                        out_hbm.at[c, pl.ds(s * rpt, rpt)])

    return k(dst_pad, zerosW, onesW)


def _sc_edge_aggregate(g, src_pad, dst_pad, zerosD, NP, D, EPW, CH):
    mesh = plsc.VectorSubcoreMesh(
        core_axis_name="c", subcore_axis_name="s", num_cores=NC, num_subcores=NS)
    rpt = NP // NS

    @functools.partial(
        pl.kernel,
        out_type=jax.ShapeDtypeStruct((NC, NP, D), jnp.float32),
        mesh=mesh,
        scratch_types=[
            pltpu.VMEM((K,), jnp.int32),
            pltpu.VMEM((K,), jnp.int32),
            pltpu.VMEM((K, D), jnp.float32),
            pltpu.SemaphoreType.DMA,
            pltpu.VMEM_SHARED((NP, D), jnp.float32),
        ],
    )
    def k(g_hbm, src_hbm, dst_hbm, zeros_hbm, out_hbm,
          sidx_v, didx_v, rows_v, sem, acc_sh):
        c = lax.axis_index("c")
        s = lax.axis_index("s")
        wid = s * NC + c
        pltpu.sync_copy(zeros_hbm.at[pl.ds(s * rpt, rpt)],
                        acc_sh.at[pl.ds(s * rpt, rpt)])
        plsc.subcore_barrier()
        e0 = pl.multiple_of(wid * EPW, 8)

        def step(i, carry):
            off = pl.multiple_of(e0 + i * K, 8)
            pltpu.sync_copy(src_hbm.at[pl.ds(off, K)], sidx_v)
            pltpu.sync_copy(dst_hbm.at[pl.ds(off, K)], didx_v)
            pltpu.async_copy(g_hbm.at[sidx_v], rows_v, sem).wait()
            pltpu.sync_copy(rows_v, acc_sh.at[didx_v], add=True)
            return carry

        lax.fori_loop(0, CH, step, 0)
        plsc.subcore_barrier()
        pltpu.sync_copy(acc_sh.at[pl.ds(s * rpt, rpt)],
                        out_hbm.at[c, pl.ds(s * rpt, rpt)])

    return k(g, src_pad, dst_pad, zerosD)


def _tc_transform(deg_parts, x_pad, W, N, NP, D):
    grid = (NP // BR,)

    def body(degp_ref, x_ref, w_ref, g_ref, dinv_ref):
        i = pl.program_id(0)
        degsum = degp_ref[0] + degp_ref[1]
        deg = degsum[:, 0:1] + 1.0
        row = lax.broadcasted_iota(jnp.int32, (BR, 1), 0) + i * BR
        dinv = jnp.where(row < N, lax.rsqrt(deg), 0.0)
        h = jnp.dot(x_ref[...], w_ref[...], preferred_element_type=jnp.float32)
        g_ref[...] = h * dinv
        dinv_ref[...] = jnp.broadcast_to(dinv, (BR, 8))

    return pl.pallas_call(
        body,
        grid=grid,
        in_specs=[
            pl.BlockSpec((NC, BR, DEGW), lambda i: (0, i, 0)),
            pl.BlockSpec((BR, D), lambda i: (i, 0)),
            pl.BlockSpec((D, D), lambda i: (0, 0)),
        ],
        out_specs=[
            pl.BlockSpec((BR, D), lambda i: (i, 0)),
            pl.BlockSpec((BR, 8), lambda i: (i, 0)),
        ],
        out_shape=[
            jax.ShapeDtypeStruct((NP, D), jnp.float32),
            jax.ShapeDtypeStruct((NP, 8), jnp.float32),
        ],
    )(deg_parts, x_pad, W)


def _tc_combine(parts, g, dinv8, b2d, NP, D):
    grid = (NP // BR,)

    def body(p_ref, g_ref, dinv_ref, b_ref, o_ref):
        ssum = p_ref[0] + p_ref[1] + g_ref[...]
        o_ref[...] = ssum * dinv_ref[:, 0:1] + b_ref[...]

    return pl.pallas_call(
        body,
        grid=grid,
        in_specs=[
            pl.BlockSpec((NC, BR, D), lambda i: (0, i, 0)),
            pl.BlockSpec((BR, D), lambda i: (i, 0)),
            pl.BlockSpec((BR, 8), lambda i: (i, 0)),
            pl.BlockSpec((1, D), lambda i: (0, 0)),
        ],
        out_specs=pl.BlockSpec((BR, D), lambda i: (i, 0)),
        out_shape=jax.ShapeDtypeStruct((NP, D), jnp.float32),
    )(parts, g, dinv8, b2d)


def kernel(x, edge_index, W, b):
    N, D_in = x.shape
    D = W.shape[1]
    E = edge_index.shape[1]

    NP = ((N + BR - 1) // BR) * BR
    EPW = ((E + NW * K - 1) // (NW * K)) * K
    EP = EPW * NW
    CH = EPW // K

    pad_e = EP - E
    src_pad = jnp.concatenate(
        [edge_index[0], jnp.full((pad_e,), N, dtype=edge_index.dtype)])
    dst_pad = jnp.concatenate(
        [edge_index[1], jnp.full((pad_e,), N, dtype=edge_index.dtype)])
    x_pad = jnp.pad(x, ((0, NP - N), (0, 0)))

    zerosD = jnp.zeros((NP, D), jnp.float32)
    onesW = jnp.ones((K, DEGW), jnp.float32)

    deg_parts = _sc_degree(dst_pad, jnp.zeros((NP, DEGW), jnp.float32), onesW, NP, EPW, CH)
    g, dinv8 = _tc_transform(deg_parts, x_pad, W, N, NP, D)
    parts = _sc_edge_aggregate(g, src_pad, dst_pad, zerosD, NP, D, EPW, CH)
    out = _tc_combine(parts, g, dinv8, b.reshape(1, D), NP, D)
    return out[:N]
